# baseline (speedup 1.0000x reference)
import functools

import jax
import jax.numpy as jnp
from jax import lax
from jax.experimental import pallas as pl
from jax.experimental.pallas import tpu as pltpu
from jax.experimental.pallas import tpu_sc as plsc

N_NODES = 100000
N_EDGES = 1600000
N_BASIS = 32
CUTOFF = 5.0

_NC = 2
_NS = 16
_L = 16
_NW = _NC * _NS
_EW = N_EDGES // _NW
_SUP = 2000
_NSUP = _EW // _SUP
_GC = 80
_NG = _SUP // _GC
_NGRP = _SUP // _L


@functools.partial(
    pl.kernel,
    out_type=[jax.ShapeDtypeStruct((N_EDGES,), jnp.float32),
              jax.ShapeDtypeStruct((N_EDGES,), jnp.float32)],
    mesh=plsc.VectorSubcoreMesh(core_axis_name="c", subcore_axis_name="s"),
    scratch_types=(
        [pltpu.VMEM((_SUP,), jnp.int32)] * 4 +
        [pltpu.VMEM((_SUP,), jnp.float32)] * 12 +
        [pltpu.VMEM((_SUP,), jnp.float32)] * 2 +
        [pltpu.SemaphoreType.DMA] * 2
    ),
)
def _sc_edge(cx_hbm, cy_hbm, cz_hbm, recv_hbm, send_hbm, t_hbm, phi_hbm,
             ridx0, sidx0, ridx1, sidx1,
             rx0, ry0, rz0, sx0, sy0, sz0,
             rx1, ry1, rz1, sx1, sy1, sz1,
             t_v, phi_v, sem0, sem1):

    wid = lax.axis_index("s") * _NC + lax.axis_index("c")
    base = wid * _EW
    set0 = (ridx0, sidx0, rx0, ry0, rz0, sx0, sy0, sz0, sem0)
    set1 = (ridx1, sidx1, rx1, ry1, rz1, sx1, sy1, sz1, sem1)

    def idxload(s, bufs):
        ridx, sidx = bufs[0], bufs[1]
        off = base + s * _SUP
        pltpu.sync_copy(recv_hbm.at[pl.ds(off, _SUP)], ridx)
        pltpu.sync_copy(send_hbm.at[pl.ds(off, _SUP)], sidx)

    def fire(bufs):
        ridx, sidx, rx, ry, rz, sx, sy, sz, sem = bufs

        def body(g, c):
            sl = pl.ds(g * _GC, _GC)
            ri = ridx.at[sl]
            si = sidx.at[sl]
            pltpu.async_copy(cx_hbm.at[ri], rx.at[sl], sem)
            pltpu.async_copy(cy_hbm.at[ri], ry.at[sl], sem)
            pltpu.async_copy(cz_hbm.at[ri], rz.at[sl], sem)
            pltpu.async_copy(cx_hbm.at[si], sx.at[sl], sem)
            pltpu.async_copy(cy_hbm.at[si], sy.at[sl], sem)
            pltpu.async_copy(cz_hbm.at[si], sz.at[sl], sem)
            return c

        lax.fori_loop(0, _NG, body, 0, unroll=False)

    def drain(bufs):
        ridx, sidx, rx, ry, rz, sx, sy, sz, sem = bufs

        def body(g, c):
            sl = pl.ds(g * _GC, _GC)
            ri = ridx.at[sl]
            si = sidx.at[sl]
            pltpu.make_async_copy(cx_hbm.at[ri], rx.at[sl], sem).wait()
            pltpu.make_async_copy(cy_hbm.at[ri], ry.at[sl], sem).wait()
            pltpu.make_async_copy(cz_hbm.at[ri], rz.at[sl], sem).wait()
            pltpu.make_async_copy(cx_hbm.at[si], sx.at[sl], sem).wait()
            pltpu.make_async_copy(cy_hbm.at[si], sy.at[sl], sem).wait()
            pltpu.make_async_copy(cz_hbm.at[si], sz.at[sl], sem).wait()
            return c

        lax.fori_loop(0, _NG, body, 0, unroll=False)

    def compstore(s, bufs):
        rx, ry, rz, sx, sy, sz = bufs[2:8]
        off = base + s * _SUP

        def body(i, c):
            sl = pl.ds(i * _L, _L)
            dx = rx[sl] - sx[sl]
            dy = ry[sl] - sy[sl]
            dz = rz[sl] - sz[sl]
            d2 = dx * dx + dy * dy + dz * dz
            ybits = jnp.int32(0x5F3759DF) - lax.shift_right_logical(
                lax.bitcast_convert_type(d2, jnp.int32), 1)
            y = lax.bitcast_convert_type(ybits, jnp.float32)
            y = y * (1.5 - 0.5 * ((d2 * y) * y))
            y = y * (1.5 - 0.5 * ((d2 * y) * y))
            y = y * (1.5 - 0.5 * ((d2 * y) * y))
            r = d2 * y
            t_v[sl] = jnp.exp(-r)
            u = r * (1.0 / CUTOFF)
            u2 = u * u
            phi_v[sl] = 1.0 + u2 * u * (-10.0 + 15.0 * u - 6.0 * u2)
            return c

        lax.fori_loop(0, _NGRP, body, 0, unroll=False)
        pltpu.sync_copy(t_v, t_hbm.at[pl.ds(off, _SUP)])
        pltpu.sync_copy(phi_v, phi_hbm.at[pl.ds(off, _SUP)])

    idxload(0, set0)
    fire(set0)
    idxload(1, set1)

    def pair_body(k, carry):
        s0 = 2 * k
        fire(set1)
        drain(set0)
        idxload(s0 + 2, set0)
        compstore(s0, set0)
        fire(set0)
        drain(set1)

        @pl.when(k < (_NSUP - 3) // 2)
        def _():
            idxload(s0 + 3, set1)

        compstore(s0 + 1, set1)
        return carry

    lax.fori_loop(0, (_NSUP - 1) // 2, pair_body, 0, unroll=False)
    drain(set0)
    compstore(_NSUP - 1, set0)


_BR = 250
_NBLK = N_EDGES // 128 // _BR
_BE = _BR * 128


def _tc_rbf(t_ref, phi_ref, mu_ref, beta_ref, out_ref):
    t3 = lax.broadcast_in_dim(t_ref[0], (_BR, 128, N_BASIS), (0, 1))
    p3 = lax.broadcast_in_dim(phi_ref[0], (_BR, 128, N_BASIS), (0, 1))
    z = t3 - mu_ref[:]
    out_ref[:] = jnp.exp(-beta_ref[:] * z * z) * p3


def kernel(coordinates, receivers, senders, mu, beta):
    coords3 = coordinates.reshape(N_NODES, 3)
    cx, cy, cz = lax.optimization_barrier(
        (coords3[:, 0], coords3[:, 1], coords3[:, 2]))
    t, phi = _sc_edge(cx, cy, cz, receivers, senders)

    rbf = pl.pallas_call(
        _tc_rbf,
        grid=(_NBLK,),
        in_specs=[
            pl.BlockSpec((1, _BR, 128), lambda i: (i, 0, 0)),
            pl.BlockSpec((1, _BR, 128), lambda i: (i, 0, 0)),
            pl.BlockSpec((1, 1, N_BASIS), lambda i: (0, 0, 0)),
            pl.BlockSpec((1, 1, N_BASIS), lambda i: (0, 0, 0)),
        ],
        out_specs=pl.BlockSpec((_BR, 128, N_BASIS), lambda i: (i, 0, 0)),
        out_shape=jax.ShapeDtypeStruct((_NBLK * _BR, 128, N_BASIS),
                                       jnp.float32),
    )(t.reshape(_NBLK, _BR, 128), phi.reshape(_NBLK, _BR, 128),
      mu.reshape(1, 1, N_BASIS), beta.reshape(1, 1, N_BASIS))
    return rbf.reshape(N_EDGES, N_BASIS)
